# Initial kernel scaffold; baseline (speedup 1.0000x reference)
import jax
import jax.numpy as jnp
from jax.experimental import pallas as pl


def kernel(x_token, x_phrase, params, ei_t2p, ei_p2t):
    raise NotImplementedError("write your pallas kernel here")


# speedup vs baseline: 6.2469x; 6.2469x over previous
import functools

import jax
import jax.numpy as jnp
from jax import lax
from jax.experimental import pallas as pl
from jax.experimental.pallas import tpu as pltpu
from jax.experimental.pallas import tpu_sc as plsc

H = 128
N = 10000
E = 320000
NC = 2
NS = 16
L = 16
SHN = 10240
C = 64
PE = 327680
EPT = PE // (NC * NS)
NCHUNK = EPT // C
RPT1 = SHN // NS
RPT2 = SHN // (NC * NS)
RB = 400
GRID = N // RB


def _emb_body(x_ref, w_ref, b_ref, o_ref):
    o_ref[...] = x_ref[...] * w_ref[...] + b_ref[...]


def _embed(x_token, w, b):
    return pl.pallas_call(
        _emb_body,
        grid=(GRID,),
        in_specs=[
            pl.BlockSpec((RB, 1), lambda i: (i, 0)),
            pl.BlockSpec((1, H), lambda i: (0, 0)),
            pl.BlockSpec((1, H), lambda i: (0, 0)),
        ],
        out_specs=pl.BlockSpec((RB, H), lambda i: (i, 0)),
        out_shape=jax.ShapeDtypeStruct((N, H), jnp.float32),
    )(x_token, w, b)


def _linear_body(x_ref, w_ref, b_ref, o_ref):
    o_ref[...] = (
        jnp.dot(x_ref[...], w_ref[...], preferred_element_type=jnp.float32)
        + b_ref[...]
    )


def _linear(x, w, b):
    dout = w.shape[1]
    return pl.pallas_call(
        _linear_body,
        grid=(GRID,),
        in_specs=[
            pl.BlockSpec((RB, H), lambda i: (i, 0)),
            pl.BlockSpec((H, dout), lambda i: (0, 0)),
            pl.BlockSpec((1, dout), lambda i: (0, 0)),
        ],
        out_specs=pl.BlockSpec((RB, dout), lambda i: (i, 0)),
        out_shape=jax.ShapeDtypeStruct((N, dout), jnp.float32),
    )(x, w, b.reshape(1, dout))


def _epi_common(out_ref, xd_ref, ws_ref, bs_ref, wba_ref, wbb_ref):
    out = out_ref[...]
    r = (
        jnp.dot(xd_ref[...], ws_ref[...], preferred_element_type=jnp.float32)
        + bs_ref[...]
    )
    z = (
        jnp.dot(out, wba_ref[...], preferred_element_type=jnp.float32)
        + jnp.dot(r, wbb_ref[...], preferred_element_type=jnp.float32)
    )
    beta = 1.0 / (1.0 + jnp.exp(-z))
    return beta * r + (1.0 - beta) * out


def _epi_leaky_body(out_ref, xd_ref, ws_ref, bs_ref, wba_ref, wbb_ref, o_ref):
    res = _epi_common(out_ref, xd_ref, ws_ref, bs_ref, wba_ref, wbb_ref)
    o_ref[...] = jnp.where(res >= 0.0, res, 0.01 * res)


def _epi_head_body(out_ref, xd_ref, ws_ref, bs_ref, wba_ref, wbb_ref,
                   hw_ref, hb_ref, o_ref):
    res = _epi_common(out_ref, xd_ref, ws_ref, bs_ref, wba_ref, wbb_ref)
    o_ref[...] = (
        jnp.dot(res, hw_ref[...], preferred_element_type=jnp.float32)
        + hb_ref[...]
    )


_EPI_SPECS = [
    pl.BlockSpec((RB, H), lambda i: (i, 0)),
    pl.BlockSpec((RB, H), lambda i: (i, 0)),
    pl.BlockSpec((H, H), lambda i: (0, 0)),
    pl.BlockSpec((1, H), lambda i: (0, 0)),
    pl.BlockSpec((H, H), lambda i: (0, 0)),
    pl.BlockSpec((H, H), lambda i: (0, 0)),
]


def _epilogue_leaky(out, x_dst, ws, bs, wba, wbb):
    return pl.pallas_call(
        _epi_leaky_body,
        grid=(GRID,),
        in_specs=_EPI_SPECS,
        out_specs=pl.BlockSpec((RB, H), lambda i: (i, 0)),
        out_shape=jax.ShapeDtypeStruct((N, H), jnp.float32),
    )(out, x_dst, ws, bs.reshape(1, H), wba, wbb)


def _epilogue_head(out, x_dst, ws, bs, wba, wbb, hw, hb):
    return pl.pallas_call(
        _epi_head_body,
        grid=(GRID,),
        in_specs=_EPI_SPECS + [
            pl.BlockSpec((H, H), lambda i: (0, 0)),
            pl.BlockSpec((1, H), lambda i: (0, 0)),
        ],
        out_specs=pl.BlockSpec((RB, H), lambda i: (i, 0)),
        out_shape=jax.ShapeDtypeStruct((N, H), jnp.float32),
    )(out, x_dst, ws, bs.reshape(1, H), wba, wbb, hw, hb.reshape(1, H))


def _take16(x, idx):
    return lax.gather(
        x, idx[:, None],
        lax.GatherDimensionNumbers(offset_dims=(), collapsed_slice_dims=(0,),
                                   start_index_map=(0,)),
        (1,), mode=lax.GatherScatterMode.PROMISE_IN_BOUNDS)


def _edge_body(q_hbm, k_hbm, v_hbm, srcg_hbm, dstg_hbm, dsts_hbm,
               out_hbm, s_hbm,
               srcv, dgv, dsv, qr, kr, vr, sloc, stmp, shared, shared_s,
               sem1, sem2, sem3):
    cid = lax.axis_index("c")
    sid = lax.axis_index("s")
    w = cid * NS + sid
    iota = lax.iota(jnp.int32, L)
    zero16 = jnp.zeros((L,), jnp.float32)

    def _zs(i, _):
        sloc[pl.ds(i * L, L)] = zero16
        return _
    lax.fori_loop(0, SHN // L, _zs, 0)

    def _zrow(r, _):
        for b in range(H // L):
            vr[r, pl.ds(b * L, L)] = zero16
        return _
    lax.fori_loop(0, C, _zrow, 0)
    for t in range(RPT1 // C):
        pltpu.sync_copy(vr, shared.at[pl.ds(sid * RPT1 + t * C, C)])
    plsc.subcore_barrier()

    def _chunk(g, _):
        start = w * EPT + g * C
        pltpu.sync_copy(srcg_hbm.at[pl.ds(start, C)], srcv)
        pltpu.sync_copy(dstg_hbm.at[pl.ds(start, C)], dgv)
        pltpu.sync_copy(dsts_hbm.at[pl.ds(start, C)], dsv)
        cp1 = pltpu.async_copy(q_hbm.at[dgv], qr, sem1)
        cp2 = pltpu.async_copy(k_hbm.at[srcv], kr, sem2)
        cp3 = pltpu.async_copy(v_hbm.at[srcv], vr, sem3)
        cp1.wait()
        cp2.wait()
        cp3.wait()

        def _group(g2, _):
            e16 = zero16
            for j in range(L):
                row = g2 * L + j
                acc = qr[row, pl.ds(0, L)] * kr[row, pl.ds(0, L)]
                for b in range(1, H // L):
                    acc = acc + (qr[row, pl.ds(b * L, L)]
                                 * kr[row, pl.ds(b * L, L)])
                for sh in (8, 4, 2, 1):
                    acc = acc + _take16(acc, (iota + sh) % L)
                e = jnp.exp(acc)
                for b in range(H // L):
                    vr[row, pl.ds(b * L, L)] = e * vr[row, pl.ds(b * L, L)]
                e16 = jnp.where(iota == j, e, e16)
            dst16 = dsv[pl.ds(g2 * L, L)]
            plsc.addupdate_scatter(sloc, [dst16], e16)
            return _
        lax.fori_loop(0, C // L, _group, 0)
        pltpu.sync_copy(vr, shared.at[dsv], add=True)
        return _

    lax.fori_loop(0, NCHUNK, _chunk, 0)

    pltpu.sync_copy(sloc, shared_s.at[pl.ds(sid * SHN, SHN)])
    plsc.subcore_barrier()

    base = sid * RPT1
    pltpu.sync_copy(shared_s.at[pl.ds(base, RPT1)], stmp.at[pl.ds(0, RPT1)])
    sacc = stmp.at[pl.ds(0, RPT1)]
    for t in range(1, NS):
        pltpu.sync_copy(shared_s.at[pl.ds(t * SHN + base, RPT1)],
                        stmp.at[pl.ds(RPT1, RPT1)])

        def _sadd(i, _):
            sacc[pl.ds(i * L, L)] = (sacc[pl.ds(i * L, L)]
                                     + stmp[pl.ds(RPT1 + i * L, L)])
            return _
        lax.fori_loop(0, RPT1 // L, _sadd, 0)
    pltpu.sync_copy(sacc, s_hbm.at[pl.ds(cid * SHN + base, RPT1)])

    for t in range(RPT1 // C):
        pltpu.sync_copy(shared.at[pl.ds(base + t * C, C)],
                        out_hbm.at[cid, pl.ds(base + t * C, C)])


@functools.lru_cache(maxsize=1)
def _edge_kernel_fn():
    return pl.kernel(
        _edge_body,
        out_type=(
            jax.ShapeDtypeStruct((NC, SHN, H), jnp.float32),
            jax.ShapeDtypeStruct((NC * SHN,), jnp.float32),
        ),
        mesh=plsc.VectorSubcoreMesh(core_axis_name="c", subcore_axis_name="s",
                                    num_cores=NC, num_subcores=NS),
        compiler_params=pltpu.CompilerParams(needs_layout_passes=False),
        scratch_types=[
            pltpu.VMEM((C,), jnp.int32),
            pltpu.VMEM((C,), jnp.int32),
            pltpu.VMEM((C,), jnp.int32),
            pltpu.VMEM((C, H), jnp.float32),
            pltpu.VMEM((C, H), jnp.float32),
            pltpu.VMEM((C, H), jnp.float32),
            pltpu.VMEM((SHN,), jnp.float32),
            pltpu.VMEM((2 * RPT1,), jnp.float32),
            pltpu.VMEM_SHARED((SHN, H), jnp.float32),
            pltpu.VMEM_SHARED((NS * SHN,), jnp.float32),
            pltpu.SemaphoreType.DMA,
            pltpu.SemaphoreType.DMA,
            pltpu.SemaphoreType.DMA,
        ],
    )


def _combine_body(part_hbm, s_hbm, out_hbm, a0, a1, sv, sem1, sem2):
    cid = lax.axis_index("c")
    sid = lax.axis_index("s")
    w = cid * NS + sid
    base = w * RPT2
    iota = lax.iota(jnp.int32, L)
    cp1 = pltpu.async_copy(part_hbm.at[0, pl.ds(base, RPT2)], a0, sem1)
    cp2 = pltpu.async_copy(part_hbm.at[1, pl.ds(base, RPT2)], a1, sem2)
    pltpu.sync_copy(s_hbm.at[pl.ds(base, RPT2)], sv.at[pl.ds(0, RPT2)])
    pltpu.sync_copy(s_hbm.at[pl.ds(SHN + base, RPT2)],
                    sv.at[pl.ds(RPT2, RPT2)])
    cp1.wait()
    cp2.wait()

    def _grp(g, _):
        stot = (sv[pl.ds(g * L, L)] + sv[pl.ds(RPT2 + g * L, L)])
        inv = 1.0 / (stot + 1e-16)
        for j in range(L):
            row = g * L + j
            bc = _take16(inv, jnp.full((L,), j, jnp.int32))
            for b in range(H // L):
                a0[row, pl.ds(b * L, L)] = bc * (a0[row, pl.ds(b * L, L)]
                                                 + a1[row, pl.ds(b * L, L)])
        return _
    lax.fori_loop(0, RPT2 // L, _grp, 0)
    pltpu.sync_copy(a0, out_hbm.at[pl.ds(base, RPT2)])


@functools.lru_cache(maxsize=1)
def _combine_kernel_fn():
    return pl.kernel(
        _combine_body,
        out_type=jax.ShapeDtypeStruct((SHN, H), jnp.float32),
        mesh=plsc.VectorSubcoreMesh(core_axis_name="c", subcore_axis_name="s",
                                    num_cores=NC, num_subcores=NS),
        compiler_params=pltpu.CompilerParams(needs_layout_passes=False),
        scratch_types=[
            pltpu.VMEM((RPT2, H), jnp.float32),
            pltpu.VMEM((RPT2, H), jnp.float32),
            pltpu.VMEM((2 * RPT2,), jnp.float32),
            pltpu.SemaphoreType.DMA,
            pltpu.SemaphoreType.DMA,
        ],
    )


def _edge_phase(q, k, v, src_g, dst_g, dst_s):
    part, s = _edge_kernel_fn()(q, k, v, src_g, dst_g, dst_s)
    return _combine_kernel_fn()(part, s)


def _conv(x_src, x_dst, src_g, dst_g, dst_s, p):
    inv = 1.0 / jnp.sqrt(jnp.float32(H))
    q = _linear(x_dst, p['Wq'] * inv, p['bq'] * inv)
    wkv = jnp.concatenate([p['Wk'], p['Wv']], axis=1)
    bkv = jnp.concatenate([p['bk'], p['bv']], axis=0)
    kv = _linear(x_src, wkv, bkv)
    k = kv[:, :H]
    v = kv[:, H:]
    out = _edge_phase(q, k, v, src_g, dst_g, dst_s)
    return out[:N]


def _gate_weights(p):
    wb = p['Wbeta']
    wba = jnp.tile(wb[:H] + wb[2 * H:], (1, H))
    wbb = jnp.tile(wb[H:2 * H] - wb[2 * H:], (1, H))
    return wba, wbb


def kernel(x_token, x_phrase, params, ei_t2p, ei_p2t):
    pad0 = jnp.zeros((PE - E,), jnp.int32)
    padn = jnp.full((PE - E,), N, jnp.int32)

    def prep(ei):
        src = ei[0].astype(jnp.int32)
        dst = ei[1].astype(jnp.int32)
        return (jnp.concatenate([src, pad0]),
                jnp.concatenate([dst, pad0]),
                jnp.concatenate([dst, padn]))

    src_t2p, dstg_t2p, dsts_t2p = prep(ei_t2p)
    src_p2t, dstg_p2t, dsts_p2t = prep(ei_p2t)

    h_t = _embed(x_token, params['emb_W'], params['emb_b'].reshape(1, H))
    h_p = x_phrase

    pt = params['t2p']
    pp = params['p2t']
    wba_t, wbb_t = _gate_weights(pt)
    wba_p, wbb_p = _gate_weights(pp)

    a = _conv(h_t, h_p, src_t2p, dstg_t2p, dsts_t2p, pt)
    h_p2 = _epilogue_leaky(a, h_p, pt['Wskip'], pt['bskip'], wba_t, wbb_t)
    b = _conv(h_p, h_t, src_p2t, dstg_p2t, dsts_p2t, pp)
    h_t2 = _epilogue_leaky(b, h_t, pp['Wskip'], pp['bskip'], wba_p, wbb_p)
    c = _conv(h_p2, h_t2, src_p2t, dstg_p2t, dsts_p2t, pp)
    return _epilogue_head(c, h_t2, pp['Wskip'], pp['bskip'], wba_p, wbb_p,
                          params['head_W'], params['head_b'])


# speedup vs baseline: 7.5985x; 1.2164x over previous
import functools

import jax
import jax.numpy as jnp
from jax import lax
from jax.experimental import pallas as pl
from jax.experimental.pallas import tpu as pltpu
from jax.experimental.pallas import tpu_sc as plsc

H = 128
N = 10000
E = 320000
NC = 2
NS = 16
L = 16
SHN = 10240
C = 32
NCHUNK = 316
EPT = C * NCHUNK
PE = EPT * NC * NS
RPT1 = SHN // NS
RPT2 = SHN // (NC * NS)
RB = 400
GRID = N // RB


def _emb_body(x_ref, w_ref, b_ref, o_ref):
    o_ref[...] = x_ref[...] * w_ref[...] + b_ref[...]


def _embed(x_token, w, b):
    return pl.pallas_call(
        _emb_body,
        grid=(GRID,),
        in_specs=[
            pl.BlockSpec((RB, 1), lambda i: (i, 0)),
            pl.BlockSpec((1, H), lambda i: (0, 0)),
            pl.BlockSpec((1, H), lambda i: (0, 0)),
        ],
        out_specs=pl.BlockSpec((RB, H), lambda i: (i, 0)),
        out_shape=jax.ShapeDtypeStruct((N, H), jnp.float32),
    )(x_token, w, b)


def _linear_body(x_ref, w_ref, b_ref, o_ref):
    o_ref[...] = (
        jnp.dot(x_ref[...], w_ref[...], preferred_element_type=jnp.float32)
        + b_ref[...]
    )


def _linear(x, w, b):
    dout = w.shape[1]
    return pl.pallas_call(
        _linear_body,
        grid=(GRID,),
        in_specs=[
            pl.BlockSpec((RB, H), lambda i: (i, 0)),
            pl.BlockSpec((H, dout), lambda i: (0, 0)),
            pl.BlockSpec((1, dout), lambda i: (0, 0)),
        ],
        out_specs=pl.BlockSpec((RB, dout), lambda i: (i, 0)),
        out_shape=jax.ShapeDtypeStruct((N, dout), jnp.float32),
    )(x, w, b.reshape(1, dout))


def _epi_common(out_ref, xd_ref, ws_ref, bs_ref, wba_ref, wbb_ref):
    out = out_ref[...]
    r = (
        jnp.dot(xd_ref[...], ws_ref[...], preferred_element_type=jnp.float32)
        + bs_ref[...]
    )
    z = (
        jnp.dot(out, wba_ref[...], preferred_element_type=jnp.float32)
        + jnp.dot(r, wbb_ref[...], preferred_element_type=jnp.float32)
    )
    beta = 1.0 / (1.0 + jnp.exp(-z))
    return beta * r + (1.0 - beta) * out


def _epi_leaky_body(out_ref, xd_ref, ws_ref, bs_ref, wba_ref, wbb_ref, o_ref):
    res = _epi_common(out_ref, xd_ref, ws_ref, bs_ref, wba_ref, wbb_ref)
    o_ref[...] = jnp.where(res >= 0.0, res, 0.01 * res)


def _epi_head_body(out_ref, xd_ref, ws_ref, bs_ref, wba_ref, wbb_ref,
                   hw_ref, hb_ref, o_ref):
    res = _epi_common(out_ref, xd_ref, ws_ref, bs_ref, wba_ref, wbb_ref)
    o_ref[...] = (
        jnp.dot(res, hw_ref[...], preferred_element_type=jnp.float32)
        + hb_ref[...]
    )


_EPI_SPECS = [
    pl.BlockSpec((RB, H), lambda i: (i, 0)),
    pl.BlockSpec((RB, H), lambda i: (i, 0)),
    pl.BlockSpec((H, H), lambda i: (0, 0)),
    pl.BlockSpec((1, H), lambda i: (0, 0)),
    pl.BlockSpec((H, H), lambda i: (0, 0)),
    pl.BlockSpec((H, H), lambda i: (0, 0)),
]


def _epilogue_leaky(out, x_dst, ws, bs, wba, wbb):
    return pl.pallas_call(
        _epi_leaky_body,
        grid=(GRID,),
        in_specs=_EPI_SPECS,
        out_specs=pl.BlockSpec((RB, H), lambda i: (i, 0)),
        out_shape=jax.ShapeDtypeStruct((N, H), jnp.float32),
    )(out, x_dst, ws, bs.reshape(1, H), wba, wbb)


def _epilogue_head(out, x_dst, ws, bs, wba, wbb, hw, hb):
    return pl.pallas_call(
        _epi_head_body,
        grid=(GRID,),
        in_specs=_EPI_SPECS + [
            pl.BlockSpec((H, H), lambda i: (0, 0)),
            pl.BlockSpec((1, H), lambda i: (0, 0)),
        ],
        out_specs=pl.BlockSpec((RB, H), lambda i: (i, 0)),
        out_shape=jax.ShapeDtypeStruct((N, H), jnp.float32),
    )(out, x_dst, ws, bs.reshape(1, H), wba, wbb, hw, hb.reshape(1, H))


def _take16(x, idx):
    return lax.gather(
        x, idx[:, None],
        lax.GatherDimensionNumbers(offset_dims=(), collapsed_slice_dims=(0,),
                                   start_index_map=(0,)),
        (1,), mode=lax.GatherScatterMode.PROMISE_IN_BOUNDS)


def _edge_body(q_hbm, k_hbm, v_hbm, srcg_hbm, dstg_hbm, dsts_hbm,
               out_hbm, s_hbm,
               srcv0, srcv1, dgv0, dgv1, dsv0, dsv1, dsw0, dsw1,
               qr0, qr1, kr0, kr1, vr0, vr1, scb0, scb1, sloc, shared,
               gsem0, gsem1, ssem0, ssem1, isem0, isem1):
    srcv = (srcv0, srcv1)
    dgv = (dgv0, dgv1)
    dsv = (dsv0, dsv1)
    dsw = (dsw0, dsw1)
    qr = (qr0, qr1)
    kr = (kr0, kr1)
    vr = (vr0, vr1)
    scb = (scb0, scb1)
    gsem = (gsem0, gsem1)
    ssem = (ssem0, ssem1)
    isem = (isem0, isem1)
    cid = lax.axis_index("c")
    sid = lax.axis_index("s")
    w = cid * NS + sid
    iota = lax.iota(jnp.int32, L)
    zero16 = jnp.zeros((L,), jnp.float32)

    def _zs(i, _):
        sloc[pl.ds(i * L, L)] = zero16
        return _
    lax.fori_loop(0, SHN // L, _zs, 0)

    def _zrow(r, _):
        for b in range(H // L):
            vr[0][r, pl.ds(b * L, L)] = zero16
        return _
    lax.fori_loop(0, C, _zrow, 0)
    for t in range(RPT1 // C):
        pltpu.sync_copy(vr[0], shared.at[pl.ds(sid * RPT1 + t * C, C)])
    plsc.subcore_barrier()

    def _fire_idx(par, g):
        start = w * EPT + g * C
        pltpu.async_copy(srcg_hbm.at[pl.ds(start, C)], srcv[par], isem[par])
        pltpu.async_copy(dstg_hbm.at[pl.ds(start, C)], dgv[par], isem[par])
        pltpu.async_copy(dsts_hbm.at[pl.ds(start, C)], dsv[par], isem[par])

    def _wait_idx(par):
        pltpu.make_async_copy(srcg_hbm.at[pl.ds(0, C)], srcv[par],
                              isem[par]).wait()
        pltpu.make_async_copy(dstg_hbm.at[pl.ds(0, C)], dgv[par],
                              isem[par]).wait()
        pltpu.make_async_copy(dsts_hbm.at[pl.ds(0, C)], dsv[par],
                              isem[par]).wait()

    def _fire_gather(par):
        pltpu.async_copy(q_hbm.at[dgv[par]], qr[par], gsem[par])
        pltpu.async_copy(k_hbm.at[srcv[par]], kr[par], gsem[par])
        pltpu.async_copy(v_hbm.at[srcv[par]], vr[par], gsem[par])

    def _wait_gather(par):
        pltpu.make_async_copy(q_hbm.at[dgv[par]], qr[par], gsem[par]).wait()
        pltpu.make_async_copy(k_hbm.at[srcv[par]], kr[par], gsem[par]).wait()
        pltpu.make_async_copy(v_hbm.at[srcv[par]], vr[par], gsem[par]).wait()

    def _wait_scatter(par):
        pltpu.make_async_copy(scb[par], shared.at[dsw[par]],
                              ssem[par]).wait()

    def _compute(par):
        def _group(g2, _):
            e16 = zero16
            for j in range(L):
                row = g2 * L + j
                acc = qr[par][row, pl.ds(0, L)] * kr[par][row, pl.ds(0, L)]
                for b in range(1, H // L):
                    acc = acc + (qr[par][row, pl.ds(b * L, L)]
                                 * kr[par][row, pl.ds(b * L, L)])
                for sh in (8, 4, 2, 1):
                    acc = acc + _take16(acc, (iota + sh) % L)
                e = jnp.exp(acc)
                for b in range(H // L):
                    scb[par][row, pl.ds(b * L, L)] = (
                        e * vr[par][row, pl.ds(b * L, L)])
                e16 = jnp.where(iota == j, e, e16)
            dst16 = dsv[par][pl.ds(g2 * L, L)]
            plsc.addupdate_scatter(sloc, [dst16], e16)
            return _
        lax.fori_loop(0, C // L, _group, 0)

    _fire_idx(0, 0)
    _fire_idx(1, 1)
    _wait_idx(0)
    _fire_gather(0)

    def _step(i, carry):
        for par in (0, 1):
            g = 2 * i + par
            _wait_gather(par)

            @pl.when(g >= 2)
            def _():
                _wait_scatter(par)
            _compute(par)
            for cc in range(C // L):
                dsw[par][pl.ds(cc * L, L)] = dsv[par][pl.ds(cc * L, L)]
            pltpu.async_copy(scb[par], shared.at[dsw[par]], ssem[par],
                             add=True)

            @pl.when(g + 2 < NCHUNK)
            def _():
                _fire_idx(par, g + 2)

            @pl.when(g + 1 < NCHUNK)
            def _():
                _wait_idx(1 - par)
                _fire_gather(1 - par)
        return carry

    lax.fori_loop(0, NCHUNK // 2, _step, 0)
    _wait_scatter(0)
    _wait_scatter(1)

    pltpu.sync_copy(sloc, s_hbm.at[pl.ds(w * SHN, SHN)])
    plsc.subcore_barrier()

    base = sid * RPT1
    for t in range(RPT1 // C):
        pltpu.sync_copy(shared.at[pl.ds(base + t * C, C)],
                        out_hbm.at[cid, pl.ds(base + t * C, C)])


@functools.lru_cache(maxsize=1)
def _edge_kernel_fn():
    return pl.kernel(
        _edge_body,
        out_type=(
            jax.ShapeDtypeStruct((NC, SHN, H), jnp.float32),
            jax.ShapeDtypeStruct((NC * NS * SHN,), jnp.float32),
        ),
        mesh=plsc.VectorSubcoreMesh(core_axis_name="c", subcore_axis_name="s",
                                    num_cores=NC, num_subcores=NS),
        compiler_params=pltpu.CompilerParams(needs_layout_passes=False),
        scratch_types=(
            [pltpu.VMEM((C,), jnp.int32)] * 8
            + [pltpu.VMEM((C, H), jnp.float32)] * 8
            + [
                pltpu.VMEM((SHN,), jnp.float32),
                pltpu.VMEM_SHARED((SHN, H), jnp.float32),
            ]
            + [pltpu.SemaphoreType.DMA] * 6
        ),
    )


def _combine_body(part_hbm, s_hbm, out_hbm, a0, a1, sv, sem1, sem2, sem3):
    cid = lax.axis_index("c")
    sid = lax.axis_index("s")
    w = cid * NS + sid
    base = w * RPT2
    iota = lax.iota(jnp.int32, L)
    cp1 = pltpu.async_copy(part_hbm.at[0, pl.ds(base, RPT2)], a0, sem1)
    cp2 = pltpu.async_copy(part_hbm.at[1, pl.ds(base, RPT2)], a1, sem2)
    for t in range(NC * NS):
        pltpu.async_copy(s_hbm.at[pl.ds(t * SHN + base, RPT2)],
                         sv.at[pl.ds(t * RPT2, RPT2)], sem3)
    for t in range(NC * NS):
        pltpu.make_async_copy(s_hbm.at[pl.ds(base, RPT2)],
                              sv.at[pl.ds(t * RPT2, RPT2)], sem3).wait()
    cp1.wait()
    cp2.wait()

    def _grp(g, _):
        stot = sv[pl.ds(g * L, L)]
        for t in range(1, NC * NS):
            stot = stot + sv[pl.ds(t * RPT2 + g * L, L)]
        inv = 1.0 / (stot + 1e-16)
        for j in range(L):
            row = g * L + j
            bc = _take16(inv, jnp.full((L,), j, jnp.int32))
            for b in range(H // L):
                a0[row, pl.ds(b * L, L)] = bc * (a0[row, pl.ds(b * L, L)]
                                                 + a1[row, pl.ds(b * L, L)])
        return _
    lax.fori_loop(0, RPT2 // L, _grp, 0)
    pltpu.sync_copy(a0, out_hbm.at[pl.ds(base, RPT2)])


@functools.lru_cache(maxsize=1)
def _combine_kernel_fn():
    return pl.kernel(
        _combine_body,
        out_type=jax.ShapeDtypeStruct((SHN, H), jnp.float32),
        mesh=plsc.VectorSubcoreMesh(core_axis_name="c", subcore_axis_name="s",
                                    num_cores=NC, num_subcores=NS),
        compiler_params=pltpu.CompilerParams(needs_layout_passes=False),
        scratch_types=[
            pltpu.VMEM((RPT2, H), jnp.float32),
            pltpu.VMEM((RPT2, H), jnp.float32),
            pltpu.VMEM((NC * NS * RPT2,), jnp.float32),
            pltpu.SemaphoreType.DMA,
            pltpu.SemaphoreType.DMA,
            pltpu.SemaphoreType.DMA,
        ],
    )


def _edge_phase(q, k, v, src_g, dst_g, dst_s):
    part, s = _edge_kernel_fn()(q, k, v, src_g, dst_g, dst_s)
    return _combine_kernel_fn()(part, s)


def _conv(x_src, x_dst, src_g, dst_g, dst_s, p):
    inv = 1.0 / jnp.sqrt(jnp.float32(H))
    q = _linear(x_dst, p['Wq'] * inv, p['bq'] * inv)
    wkv = jnp.concatenate([p['Wk'], p['Wv']], axis=1)
    bkv = jnp.concatenate([p['bk'], p['bv']], axis=0)
    kv = _linear(x_src, wkv, bkv)
    k = kv[:, :H]
    v = kv[:, H:]
    out = _edge_phase(q, k, v, src_g, dst_g, dst_s)
    return out[:N]


def _gate_weights(p):
    wb = p['Wbeta']
    wba = jnp.tile(wb[:H] + wb[2 * H:], (1, H))
    wbb = jnp.tile(wb[H:2 * H] - wb[2 * H:], (1, H))
    return wba, wbb


def kernel(x_token, x_phrase, params, ei_t2p, ei_p2t):
    pad0 = jnp.zeros((PE - E,), jnp.int32)
    padn = jnp.full((PE - E,), N, jnp.int32)

    def prep(ei):
        src = ei[0].astype(jnp.int32)
        dst = ei[1].astype(jnp.int32)
        return (jnp.concatenate([src, pad0]),
                jnp.concatenate([dst, pad0]),
                jnp.concatenate([dst, padn]))

    src_t2p, dstg_t2p, dsts_t2p = prep(ei_t2p)
    src_p2t, dstg_p2t, dsts_p2t = prep(ei_p2t)

    h_t = _embed(x_token, params['emb_W'], params['emb_b'].reshape(1, H))
    h_p = x_phrase

    pt = params['t2p']
    pp = params['p2t']
    wba_t, wbb_t = _gate_weights(pt)
    wba_p, wbb_p = _gate_weights(pp)

    a = _conv(h_t, h_p, src_t2p, dstg_t2p, dsts_t2p, pt)
    h_p2 = _epilogue_leaky(a, h_p, pt['Wskip'], pt['bskip'], wba_t, wbb_t)
    b = _conv(h_p, h_t, src_p2t, dstg_p2t, dsts_p2t, pp)
    h_t2 = _epilogue_leaky(b, h_t, pp['Wskip'], pp['bskip'], wba_p, wbb_p)
    c = _conv(h_p2, h_t2, src_p2t, dstg_p2t, dsts_p2t, pp)
    return _epilogue_head(c, h_t2, pp['Wskip'], pp['bskip'], wba_p, wbb_p,
                          params['head_W'], params['head_b'])


# speedup vs baseline: 9.3108x; 1.2254x over previous
import functools

import jax
import jax.numpy as jnp
from jax import lax
from jax.experimental import pallas as pl
from jax.experimental.pallas import tpu as pltpu
from jax.experimental.pallas import tpu_sc as plsc

H = 128
N = 10000
E = 320000
NC = 2
NS = 16
L = 16
SHN = 10240
C = 16
NCHUNK = 627
EPT = C * NCHUNK
PE = EPT * NC * NS
RPT1 = SHN // NS
RPT2 = SHN // (NC * NS)
RB = 400
GRID = N // RB


def _emb_body(x_ref, w_ref, b_ref, o_ref):
    o_ref[...] = x_ref[...] * w_ref[...] + b_ref[...]


def _embed(x_token, w, b):
    return pl.pallas_call(
        _emb_body,
        grid=(GRID,),
        in_specs=[
            pl.BlockSpec((RB, 1), lambda i: (i, 0)),
            pl.BlockSpec((1, H), lambda i: (0, 0)),
            pl.BlockSpec((1, H), lambda i: (0, 0)),
        ],
        out_specs=pl.BlockSpec((RB, H), lambda i: (i, 0)),
        out_shape=jax.ShapeDtypeStruct((N, H), jnp.float32),
    )(x_token, w, b)


def _linear_body(x_ref, w_ref, b_ref, o_ref):
    o_ref[...] = (
        jnp.dot(x_ref[...], w_ref[...], preferred_element_type=jnp.float32)
        + b_ref[...]
    )


def _linear(x, w, b):
    dout = w.shape[1]
    return pl.pallas_call(
        _linear_body,
        grid=(GRID,),
        in_specs=[
            pl.BlockSpec((RB, H), lambda i: (i, 0)),
            pl.BlockSpec((H, dout), lambda i: (0, 0)),
            pl.BlockSpec((1, dout), lambda i: (0, 0)),
        ],
        out_specs=pl.BlockSpec((RB, dout), lambda i: (i, 0)),
        out_shape=jax.ShapeDtypeStruct((N, dout), jnp.float32),
    )(x, w, b.reshape(1, dout))


def _epi_common(out_ref, xd_ref, ws_ref, bs_ref, wba_ref, wbb_ref):
    out = out_ref[...]
    r = (
        jnp.dot(xd_ref[...], ws_ref[...], preferred_element_type=jnp.float32)
        + bs_ref[...]
    )
    z = (
        jnp.dot(out, wba_ref[...], preferred_element_type=jnp.float32)
        + jnp.dot(r, wbb_ref[...], preferred_element_type=jnp.float32)
    )
    beta = 1.0 / (1.0 + jnp.exp(-z))
    return beta * r + (1.0 - beta) * out


def _epi_leaky_body(out_ref, xd_ref, ws_ref, bs_ref, wba_ref, wbb_ref, o_ref):
    res = _epi_common(out_ref, xd_ref, ws_ref, bs_ref, wba_ref, wbb_ref)
    o_ref[...] = jnp.where(res >= 0.0, res, 0.01 * res)


def _epi_head_body(out_ref, xd_ref, ws_ref, bs_ref, wba_ref, wbb_ref,
                   hw_ref, hb_ref, o_ref):
    res = _epi_common(out_ref, xd_ref, ws_ref, bs_ref, wba_ref, wbb_ref)
    o_ref[...] = (
        jnp.dot(res, hw_ref[...], preferred_element_type=jnp.float32)
        + hb_ref[...]
    )


_EPI_SPECS = [
    pl.BlockSpec((RB, H), lambda i: (i, 0)),
    pl.BlockSpec((RB, H), lambda i: (i, 0)),
    pl.BlockSpec((H, H), lambda i: (0, 0)),
    pl.BlockSpec((1, H), lambda i: (0, 0)),
    pl.BlockSpec((H, H), lambda i: (0, 0)),
    pl.BlockSpec((H, H), lambda i: (0, 0)),
]


def _epilogue_leaky(out, x_dst, ws, bs, wba, wbb):
    return pl.pallas_call(
        _epi_leaky_body,
        grid=(GRID,),
        in_specs=_EPI_SPECS,
        out_specs=pl.BlockSpec((RB, H), lambda i: (i, 0)),
        out_shape=jax.ShapeDtypeStruct((N, H), jnp.float32),
    )(out, x_dst, ws, bs.reshape(1, H), wba, wbb)


def _epilogue_head(out, x_dst, ws, bs, wba, wbb, hw, hb):
    return pl.pallas_call(
        _epi_head_body,
        grid=(GRID,),
        in_specs=_EPI_SPECS + [
            pl.BlockSpec((H, H), lambda i: (0, 0)),
            pl.BlockSpec((1, H), lambda i: (0, 0)),
        ],
        out_specs=pl.BlockSpec((RB, H), lambda i: (i, 0)),
        out_shape=jax.ShapeDtypeStruct((N, H), jnp.float32),
    )(out, x_dst, ws, bs.reshape(1, H), wba, wbb, hw, hb.reshape(1, H))


def _take16(x, idx):
    return lax.gather(
        x, idx[:, None],
        lax.GatherDimensionNumbers(offset_dims=(), collapsed_slice_dims=(0,),
                                   start_index_map=(0,)),
        (1,), mode=lax.GatherScatterMode.PROMISE_IN_BOUNDS)


def _edge_body(q_hbm, k_hbm, v_hbm, srcg_hbm, dstg_hbm, dsts_hbm,
               out_hbm, s_hbm,
               srcv0, srcv1, srcv2, dgv0, dgv1, dgv2, dsv0, dsv1, dsv2,
               dsw0, dsw1, dsw2,
               qr0, qr1, qr2, kr0, kr1, kr2, vr0, vr1, vr2, sloc, shared,
               gsem0, gsem1, gsem2, ssem0, ssem1, ssem2,
               isem0, isem1, isem2):
    srcv = (srcv0, srcv1, srcv2)
    dgv = (dgv0, dgv1, dgv2)
    dsv = (dsv0, dsv1, dsv2)
    dsw = (dsw0, dsw1, dsw2)
    qr = (qr0, qr1, qr2)
    kr = (kr0, kr1, kr2)
    vr = (vr0, vr1, vr2)
    gsem = (gsem0, gsem1, gsem2)
    ssem = (ssem0, ssem1, ssem2)
    isem = (isem0, isem1, isem2)
    cid = lax.axis_index("c")
    sid = lax.axis_index("s")
    w = cid * NS + sid
    iota = lax.iota(jnp.int32, L)
    zero16 = jnp.zeros((L,), jnp.float32)

    def _zs(i, _):
        sloc[pl.ds(i * L, L)] = zero16
        return _
    lax.fori_loop(0, SHN // L, _zs, 0)

    def _zrow(r, _):
        for b in range(H // L):
            vr[0][r, pl.ds(b * L, L)] = zero16
        return _
    lax.fori_loop(0, C, _zrow, 0)
    for t in range(RPT1 // C):
        pltpu.sync_copy(vr[0], shared.at[pl.ds(sid * RPT1 + t * C, C)])
    plsc.subcore_barrier()

    def _fire_idx(par, g):
        start = w * EPT + g * C
        pltpu.async_copy(srcg_hbm.at[pl.ds(start, C)], srcv[par], isem[par])
        pltpu.async_copy(dstg_hbm.at[pl.ds(start, C)], dgv[par], isem[par])
        pltpu.async_copy(dsts_hbm.at[pl.ds(start, C)], dsv[par], isem[par])

    def _wait_idx(par):
        pltpu.make_async_copy(srcg_hbm.at[pl.ds(0, C)], srcv[par],
                              isem[par]).wait()
        pltpu.make_async_copy(dstg_hbm.at[pl.ds(0, C)], dgv[par],
                              isem[par]).wait()
        pltpu.make_async_copy(dsts_hbm.at[pl.ds(0, C)], dsv[par],
                              isem[par]).wait()

    def _fire_gather(s):
        pltpu.async_copy(q_hbm.at[dgv[s]], qr[s], gsem[s])
        pltpu.async_copy(k_hbm.at[srcv[s]], kr[s], gsem[s])
        pltpu.async_copy(v_hbm.at[srcv[s]], vr[s], gsem[s])

    def _wait_gather(s):
        pltpu.make_async_copy(q_hbm.at[dgv[s]], qr[s], gsem[s]).wait()
        pltpu.make_async_copy(k_hbm.at[srcv[s]], kr[s], gsem[s]).wait()
        pltpu.make_async_copy(v_hbm.at[srcv[s]], vr[s], gsem[s]).wait()

    def _wait_scatter(s):
        pltpu.make_async_copy(vr[s], shared.at[dsw[s]], ssem[s]).wait()

    def _compute(s):
        def _group(g2, _):
            e16 = zero16
            for j in range(L):
                row = g2 * L + j
                acc = qr[s][row, pl.ds(0, L)] * kr[s][row, pl.ds(0, L)]
                for b in range(1, H // L):
                    acc = acc + (qr[s][row, pl.ds(b * L, L)]
                                 * kr[s][row, pl.ds(b * L, L)])
                for sh in (8, 4, 2, 1):
                    acc = acc + _take16(acc, (iota + sh) % L)
                e = jnp.exp(acc)
                for b in range(H // L):
                    vr[s][row, pl.ds(b * L, L)] = (
                        e * vr[s][row, pl.ds(b * L, L)])
                e16 = jnp.where(iota == j, e, e16)
            dst16 = dsv[s][pl.ds(g2 * L, L)]
            plsc.addupdate_scatter(sloc, [dst16], e16)
            return _
        lax.fori_loop(0, C // L, _group, 0)

    _fire_idx(0, 0)
    _fire_idx(1, 1)
    _wait_idx(0)
    _fire_gather(0)

    def _step(i, carry):
        for g3 in range(3):
            g = 3 * i + g3
            s = g3
            s1 = (g3 + 1) % 3
            s2 = (g3 + 2) % 3

            @pl.when(g + 1 < NCHUNK)
            def _():
                _wait_idx(s1)

            @pl.when(g >= 2)
            def _():
                _wait_scatter(s1)

            @pl.when(g + 1 < NCHUNK)
            def _():
                _fire_gather(s1)
            _wait_gather(s)
            _compute(s)
            for cc in range(C // L):
                dsw[s][pl.ds(cc * L, L)] = dsv[s][pl.ds(cc * L, L)]
            pltpu.async_copy(vr[s], shared.at[dsw[s]], ssem[s],
                             add=True)

            @pl.when(g + 2 < NCHUNK)
            def _():
                _fire_idx(s2, g + 2)
        return carry

    lax.fori_loop(0, NCHUNK // 3, _step, 0)
    _wait_scatter(1)
    _wait_scatter(2)

    pltpu.sync_copy(sloc, s_hbm.at[pl.ds(w * SHN, SHN)])
    plsc.subcore_barrier()

    base = sid * RPT1
    for t in range(RPT1 // C):
        pltpu.sync_copy(shared.at[pl.ds(base + t * C, C)],
                        out_hbm.at[cid, pl.ds(base + t * C, C)])


@functools.lru_cache(maxsize=1)
def _edge_kernel_fn():
    return pl.kernel(
        _edge_body,
        out_type=(
            jax.ShapeDtypeStruct((NC, SHN, H), jnp.float32),
            jax.ShapeDtypeStruct((NC * NS * SHN,), jnp.float32),
        ),
        mesh=plsc.VectorSubcoreMesh(core_axis_name="c", subcore_axis_name="s",
                                    num_cores=NC, num_subcores=NS),
        compiler_params=pltpu.CompilerParams(needs_layout_passes=False),
        scratch_types=(
            [pltpu.VMEM((C,), jnp.int32)] * 12
            + [pltpu.VMEM((C, H), jnp.float32)] * 9
            + [
                pltpu.VMEM((SHN,), jnp.float32),
                pltpu.VMEM_SHARED((SHN, H), jnp.float32),
            ]
            + [pltpu.SemaphoreType.DMA] * 9
        ),
    )


def _combine_body(part_hbm, s_hbm, out_hbm, a0, a1, sv, sem1, sem2, sem3):
    cid = lax.axis_index("c")
    sid = lax.axis_index("s")
    w = cid * NS + sid
    base = w * RPT2
    iota = lax.iota(jnp.int32, L)
    cp1 = pltpu.async_copy(part_hbm.at[0, pl.ds(base, RPT2)], a0, sem1)
    cp2 = pltpu.async_copy(part_hbm.at[1, pl.ds(base, RPT2)], a1, sem2)
    for t in range(NC * NS):
        pltpu.async_copy(s_hbm.at[pl.ds(t * SHN + base, RPT2)],
                         sv.at[pl.ds(t * RPT2, RPT2)], sem3)
    for t in range(NC * NS):
        pltpu.make_async_copy(s_hbm.at[pl.ds(base, RPT2)],
                              sv.at[pl.ds(t * RPT2, RPT2)], sem3).wait()
    cp1.wait()
    cp2.wait()

    def _grp(g, _):
        stot = sv[pl.ds(g * L, L)]
        for t in range(1, NC * NS):
            stot = stot + sv[pl.ds(t * RPT2 + g * L, L)]
        inv = 1.0 / (stot + 1e-16)
        for j in range(L):
            row = g * L + j
            bc = _take16(inv, jnp.full((L,), j, jnp.int32))
            for b in range(H // L):
                a0[row, pl.ds(b * L, L)] = bc * (a0[row, pl.ds(b * L, L)]
                                                 + a1[row, pl.ds(b * L, L)])
        return _
    lax.fori_loop(0, RPT2 // L, _grp, 0)
    pltpu.sync_copy(a0, out_hbm.at[pl.ds(base, RPT2)])


@functools.lru_cache(maxsize=1)
def _combine_kernel_fn():
    return pl.kernel(
        _combine_body,
        out_type=jax.ShapeDtypeStruct((SHN, H), jnp.float32),
        mesh=plsc.VectorSubcoreMesh(core_axis_name="c", subcore_axis_name="s",
                                    num_cores=NC, num_subcores=NS),
        compiler_params=pltpu.CompilerParams(needs_layout_passes=False),
        scratch_types=[
            pltpu.VMEM((RPT2, H), jnp.float32),
            pltpu.VMEM((RPT2, H), jnp.float32),
            pltpu.VMEM((NC * NS * RPT2,), jnp.float32),
            pltpu.SemaphoreType.DMA,
            pltpu.SemaphoreType.DMA,
            pltpu.SemaphoreType.DMA,
        ],
    )


def _edge_phase(q, k, v, src_g, dst_g, dst_s):
    part, s = _edge_kernel_fn()(q, k, v, src_g, dst_g, dst_s)
    return _combine_kernel_fn()(part, s)


def _conv(x_src, x_dst, src_g, dst_g, dst_s, p):
    inv = 1.0 / jnp.sqrt(jnp.float32(H))
    q = _linear(x_dst, p['Wq'] * inv, p['bq'] * inv)
    wkv = jnp.concatenate([p['Wk'], p['Wv']], axis=1)
    bkv = jnp.concatenate([p['bk'], p['bv']], axis=0)
    kv = _linear(x_src, wkv, bkv)
    k = kv[:, :H]
    v = kv[:, H:]
    out = _edge_phase(q, k, v, src_g, dst_g, dst_s)
    return out[:N]


def _gate_weights(p):
    wb = p['Wbeta']
    wba = jnp.tile(wb[:H] + wb[2 * H:], (1, H))
    wbb = jnp.tile(wb[H:2 * H] - wb[2 * H:], (1, H))
    return wba, wbb


def kernel(x_token, x_phrase, params, ei_t2p, ei_p2t):
    pad0 = jnp.zeros((PE - E,), jnp.int32)
    padn = jnp.full((PE - E,), N, jnp.int32)

    def prep(ei):
        src = ei[0].astype(jnp.int32)
        dst = ei[1].astype(jnp.int32)
        return (jnp.concatenate([src, pad0]),
                jnp.concatenate([dst, pad0]),
                jnp.concatenate([dst, padn]))

    src_t2p, dstg_t2p, dsts_t2p = prep(ei_t2p)
    src_p2t, dstg_p2t, dsts_p2t = prep(ei_p2t)

    h_t = _embed(x_token, params['emb_W'], params['emb_b'].reshape(1, H))
    h_p = x_phrase

    pt = params['t2p']
    pp = params['p2t']
    wba_t, wbb_t = _gate_weights(pt)
    wba_p, wbb_p = _gate_weights(pp)

    a = _conv(h_t, h_p, src_t2p, dstg_t2p, dsts_t2p, pt)
    h_p2 = _epilogue_leaky(a, h_p, pt['Wskip'], pt['bskip'], wba_t, wbb_t)
    b = _conv(h_p, h_t, src_p2t, dstg_p2t, dsts_p2t, pp)
    h_t2 = _epilogue_leaky(b, h_t, pp['Wskip'], pp['bskip'], wba_p, wbb_p)
    c = _conv(h_p2, h_t2, src_p2t, dstg_p2t, dsts_p2t, pp)
    return _epilogue_head(c, h_t2, pp['Wskip'], pp['bskip'], wba_p, wbb_p,
                          params['head_W'], params['head_b'])


# speedup vs baseline: 9.5748x; 1.0283x over previous
import functools

import jax
import jax.numpy as jnp
from jax import lax
from jax.experimental import pallas as pl
from jax.experimental.pallas import tpu as pltpu
from jax.experimental.pallas import tpu_sc as plsc

H = 128
N = 10000
E = 320000
NC = 2
NS = 16
L = 16
SHN = 10240
C = 16
NCHUNK = 627
EPT = C * NCHUNK
PE = EPT * NC * NS
RPT1 = SHN // NS
RPT2 = SHN // (NC * NS)
RB = 400
GRID = N // RB


def _emb_body(x_ref, w_ref, b_ref, o_ref):
    o_ref[...] = x_ref[...] * w_ref[...] + b_ref[...]


def _embed(x_token, w, b):
    return pl.pallas_call(
        _emb_body,
        grid=(GRID,),
        in_specs=[
            pl.BlockSpec((RB, 1), lambda i: (i, 0)),
            pl.BlockSpec((1, H), lambda i: (0, 0)),
            pl.BlockSpec((1, H), lambda i: (0, 0)),
        ],
        out_specs=pl.BlockSpec((RB, H), lambda i: (i, 0)),
        out_shape=jax.ShapeDtypeStruct((N, H), jnp.float32),
    )(x_token, w, b)


def _linear_body(x_ref, w_ref, b_ref, o_ref):
    o_ref[...] = (
        jnp.dot(x_ref[...], w_ref[...], preferred_element_type=jnp.float32)
        + b_ref[...]
    )


def _linear(x, w, b):
    dout = w.shape[1]
    return pl.pallas_call(
        _linear_body,
        grid=(GRID,),
        in_specs=[
            pl.BlockSpec((RB, H), lambda i: (i, 0)),
            pl.BlockSpec((H, dout), lambda i: (0, 0)),
            pl.BlockSpec((1, dout), lambda i: (0, 0)),
        ],
        out_specs=pl.BlockSpec((RB, dout), lambda i: (i, 0)),
        out_shape=jax.ShapeDtypeStruct((N, dout), jnp.float32),
    )(x, w, b.reshape(1, dout))


def _epi_common(out_ref, xd_ref, ws_ref, bs_ref, wba_ref, wbb_ref):
    out = out_ref[...]
    r = (
        jnp.dot(xd_ref[...], ws_ref[...], preferred_element_type=jnp.float32)
        + bs_ref[...]
    )
    z = (
        jnp.dot(out, wba_ref[...], preferred_element_type=jnp.float32)
        + jnp.dot(r, wbb_ref[...], preferred_element_type=jnp.float32)
    )
    beta = 1.0 / (1.0 + jnp.exp(-z))
    return beta * r + (1.0 - beta) * out


def _epi_leaky_body(out_ref, xd_ref, ws_ref, bs_ref, wba_ref, wbb_ref, o_ref):
    res = _epi_common(out_ref, xd_ref, ws_ref, bs_ref, wba_ref, wbb_ref)
    o_ref[...] = jnp.where(res >= 0.0, res, 0.01 * res)


def _epi_head_body(out_ref, xd_ref, ws_ref, bs_ref, wba_ref, wbb_ref,
                   hw_ref, hb_ref, o_ref):
    res = _epi_common(out_ref, xd_ref, ws_ref, bs_ref, wba_ref, wbb_ref)
    o_ref[...] = (
        jnp.dot(res, hw_ref[...], preferred_element_type=jnp.float32)
        + hb_ref[...]
    )


_EPI_SPECS = [
    pl.BlockSpec((RB, H), lambda i: (i, 0)),
    pl.BlockSpec((RB, H), lambda i: (i, 0)),
    pl.BlockSpec((H, H), lambda i: (0, 0)),
    pl.BlockSpec((1, H), lambda i: (0, 0)),
    pl.BlockSpec((H, H), lambda i: (0, 0)),
    pl.BlockSpec((H, H), lambda i: (0, 0)),
]


def _epilogue_leaky(out, x_dst, ws, bs, wba, wbb):
    return pl.pallas_call(
        _epi_leaky_body,
        grid=(GRID,),
        in_specs=_EPI_SPECS,
        out_specs=pl.BlockSpec((RB, H), lambda i: (i, 0)),
        out_shape=jax.ShapeDtypeStruct((N, H), jnp.float32),
    )(out, x_dst, ws, bs.reshape(1, H), wba, wbb)


def _epilogue_head(out, x_dst, ws, bs, wba, wbb, hw, hb):
    return pl.pallas_call(
        _epi_head_body,
        grid=(GRID,),
        in_specs=_EPI_SPECS + [
            pl.BlockSpec((H, H), lambda i: (0, 0)),
            pl.BlockSpec((1, H), lambda i: (0, 0)),
        ],
        out_specs=pl.BlockSpec((RB, H), lambda i: (i, 0)),
        out_shape=jax.ShapeDtypeStruct((N, H), jnp.float32),
    )(out, x_dst, ws, bs.reshape(1, H), wba, wbb, hw, hb.reshape(1, H))


def _take16(x, idx):
    return lax.gather(
        x, idx[:, None],
        lax.GatherDimensionNumbers(offset_dims=(), collapsed_slice_dims=(0,),
                                   start_index_map=(0,)),
        (1,), mode=lax.GatherScatterMode.PROMISE_IN_BOUNDS)


def _edge_body(q_hbm, kv_hbm, srcg_hbm, dstg_hbm, dsts_hbm,
               out_hbm, s_hbm,
               srcv0, srcv1, srcv2, dgv0, dgv1, dgv2, dsv0, dsv1, dsv2,
               dsw0, dsw1, dsw2,
               qr0, qr1, qr2, kvr0, kvr1, kvr2, scb0, scb1, scb2,
               sloc, shared,
               gsem0, gsem1, gsem2, ssem0, ssem1, ssem2,
               isem0, isem1, isem2):
    srcv = (srcv0, srcv1, srcv2)
    dgv = (dgv0, dgv1, dgv2)
    dsv = (dsv0, dsv1, dsv2)
    dsw = (dsw0, dsw1, dsw2)
    qr = (qr0, qr1, qr2)
    kvr = (kvr0, kvr1, kvr2)
    scb = (scb0, scb1, scb2)
    gsem = (gsem0, gsem1, gsem2)
    ssem = (ssem0, ssem1, ssem2)
    isem = (isem0, isem1, isem2)
    cid = lax.axis_index("c")
    sid = lax.axis_index("s")
    w = cid * NS + sid
    iota = lax.iota(jnp.int32, L)
    zero16 = jnp.zeros((L,), jnp.float32)

    def _zs(i, _):
        sloc[pl.ds(i * L, L)] = zero16
        return _
    lax.fori_loop(0, SHN // L, _zs, 0)

    def _zrow(r, _):
        for b in range(H // L):
            scb[0][r, pl.ds(b * L, L)] = zero16
        return _
    lax.fori_loop(0, C, _zrow, 0)
    for t in range(RPT1 // C):
        pltpu.sync_copy(scb[0], shared.at[pl.ds(sid * RPT1 + t * C, C)])
    plsc.subcore_barrier()

    def _fire_idx(par, g):
        start = w * EPT + g * C
        pltpu.async_copy(srcg_hbm.at[pl.ds(start, C)], srcv[par], isem[par])
        pltpu.async_copy(dstg_hbm.at[pl.ds(start, C)], dgv[par], isem[par])
        pltpu.async_copy(dsts_hbm.at[pl.ds(start, C)], dsv[par], isem[par])

    def _wait_idx(par):
        pltpu.make_async_copy(srcg_hbm.at[pl.ds(0, C)], srcv[par],
                              isem[par]).wait()
        pltpu.make_async_copy(dstg_hbm.at[pl.ds(0, C)], dgv[par],
                              isem[par]).wait()
        pltpu.make_async_copy(dsts_hbm.at[pl.ds(0, C)], dsv[par],
                              isem[par]).wait()

    def _fire_gather(s):
        pltpu.async_copy(q_hbm.at[dgv[s]], qr[s], gsem[s])
        pltpu.async_copy(kv_hbm.at[srcv[s]], kvr[s], gsem[s])

    def _wait_gather(s):
        pltpu.make_async_copy(q_hbm.at[dgv[s]], qr[s], gsem[s]).wait()
        pltpu.make_async_copy(kv_hbm.at[srcv[s]], kvr[s], gsem[s]).wait()

    def _wait_scatter(s):
        pltpu.make_async_copy(scb[s], shared.at[dsw[s]], ssem[s]).wait()

    def _compute(s):
        def _group(g2, _):
            e16 = zero16
            for j in range(L):
                row = g2 * L + j
                acc = qr[s][row, pl.ds(0, L)] * kvr[s][row, pl.ds(0, L)]
                for b in range(1, H // L):
                    acc = acc + (qr[s][row, pl.ds(b * L, L)]
                                 * kvr[s][row, pl.ds(b * L, L)])
                for sh in (8, 4, 2, 1):
                    acc = acc + _take16(acc, (iota + sh) % L)
                e = jnp.exp(acc)
                for b in range(H // L):
                    scb[s][row, pl.ds(b * L, L)] = (
                        e * kvr[s][row, pl.ds(H + b * L, L)])
                e16 = jnp.where(iota == j, e, e16)
            dst16 = dsv[s][pl.ds(g2 * L, L)]
            plsc.addupdate_scatter(sloc, [dst16], e16)
            return _
        lax.fori_loop(0, C // L, _group, 0)

    _fire_idx(0, 0)
    _fire_idx(1, 1)
    _wait_idx(0)
    _fire_gather(0)

    def _step(i, carry):
        for g3 in range(3):
            g = 3 * i + g3
            s = g3
            s1 = (g3 + 1) % 3
            s2 = (g3 + 2) % 3

            @pl.when(g + 1 < NCHUNK)
            def _():
                _wait_idx(s1)

            @pl.when(g >= 2)
            def _():
                _wait_scatter(s1)

            @pl.when(g + 1 < NCHUNK)
            def _():
                _fire_gather(s1)
            _wait_gather(s)
            _compute(s)
            for cc in range(C // L):
                dsw[s][pl.ds(cc * L, L)] = dsv[s][pl.ds(cc * L, L)]
            pltpu.async_copy(scb[s], shared.at[dsw[s]], ssem[s],
                             add=True)

            @pl.when(g + 2 < NCHUNK)
            def _():
                _fire_idx(s2, g + 2)
        return carry

    lax.fori_loop(0, NCHUNK // 3, _step, 0)
    _wait_scatter(1)
    _wait_scatter(2)

    pltpu.sync_copy(sloc, s_hbm.at[pl.ds(w * SHN, SHN)])
    plsc.subcore_barrier()

    base = sid * RPT1
    for t in range(RPT1 // C):
        pltpu.sync_copy(shared.at[pl.ds(base + t * C, C)],
                        out_hbm.at[cid, pl.ds(base + t * C, C)])


@functools.lru_cache(maxsize=1)
def _edge_kernel_fn():
    return pl.kernel(
        _edge_body,
        out_type=(
            jax.ShapeDtypeStruct((NC, SHN, H), jnp.float32),
            jax.ShapeDtypeStruct((NC * NS * SHN,), jnp.float32),
        ),
        mesh=plsc.VectorSubcoreMesh(core_axis_name="c", subcore_axis_name="s",
                                    num_cores=NC, num_subcores=NS),
        compiler_params=pltpu.CompilerParams(needs_layout_passes=False),
        scratch_types=(
            [pltpu.VMEM((C,), jnp.int32)] * 12
            + [pltpu.VMEM((C, H), jnp.float32)] * 3
            + [pltpu.VMEM((C, 2 * H), jnp.float32)] * 3
            + [pltpu.VMEM((C, H), jnp.float32)] * 3
            + [
                pltpu.VMEM((SHN,), jnp.float32),
                pltpu.VMEM_SHARED((SHN, H), jnp.float32),
            ]
            + [pltpu.SemaphoreType.DMA] * 9
        ),
    )


def _combine_body(part_hbm, s_hbm, out_hbm, a0, a1, sv, sem1, sem2, sem3):
    cid = lax.axis_index("c")
    sid = lax.axis_index("s")
    w = cid * NS + sid
    base = w * RPT2
    iota = lax.iota(jnp.int32, L)
    cp1 = pltpu.async_copy(part_hbm.at[0, pl.ds(base, RPT2)], a0, sem1)
    cp2 = pltpu.async_copy(part_hbm.at[1, pl.ds(base, RPT2)], a1, sem2)
    for t in range(NC * NS):
        pltpu.async_copy(s_hbm.at[pl.ds(t * SHN + base, RPT2)],
                         sv.at[pl.ds(t * RPT2, RPT2)], sem3)
    for t in range(NC * NS):
        pltpu.make_async_copy(s_hbm.at[pl.ds(base, RPT2)],
                              sv.at[pl.ds(t * RPT2, RPT2)], sem3).wait()
    cp1.wait()
    cp2.wait()

    def _grp(g, _):
        stot = sv[pl.ds(g * L, L)]
        for t in range(1, NC * NS):
            stot = stot + sv[pl.ds(t * RPT2 + g * L, L)]
        inv = 1.0 / (stot + 1e-16)
        for j in range(L):
            row = g * L + j
            bc = _take16(inv, jnp.full((L,), j, jnp.int32))
            for b in range(H // L):
                a0[row, pl.ds(b * L, L)] = bc * (a0[row, pl.ds(b * L, L)]
                                                 + a1[row, pl.ds(b * L, L)])
        return _
    lax.fori_loop(0, RPT2 // L, _grp, 0)
    pltpu.sync_copy(a0, out_hbm.at[pl.ds(base, RPT2)])


@functools.lru_cache(maxsize=1)
def _combine_kernel_fn():
    return pl.kernel(
        _combine_body,
        out_type=jax.ShapeDtypeStruct((SHN, H), jnp.float32),
        mesh=plsc.VectorSubcoreMesh(core_axis_name="c", subcore_axis_name="s",
                                    num_cores=NC, num_subcores=NS),
        compiler_params=pltpu.CompilerParams(needs_layout_passes=False),
        scratch_types=[
            pltpu.VMEM((RPT2, H), jnp.float32),
            pltpu.VMEM((RPT2, H), jnp.float32),
            pltpu.VMEM((NC * NS * RPT2,), jnp.float32),
            pltpu.SemaphoreType.DMA,
            pltpu.SemaphoreType.DMA,
            pltpu.SemaphoreType.DMA,
        ],
    )


def _edge_phase(q, kv, src_g, dst_g, dst_s):
    part, s = _edge_kernel_fn()(q, kv, src_g, dst_g, dst_s)
    return _combine_kernel_fn()(part, s)


def _conv(x_src, x_dst, src_g, dst_g, dst_s, p):
    inv = 1.0 / jnp.sqrt(jnp.float32(H))
    q = _linear(x_dst, p['Wq'] * inv, p['bq'] * inv)
    wkv = jnp.concatenate([p['Wk'], p['Wv']], axis=1)
    bkv = jnp.concatenate([p['bk'], p['bv']], axis=0)
    kv = _linear(x_src, wkv, bkv)
    out = _edge_phase(q, kv, src_g, dst_g, dst_s)
    return out[:N]


def _gate_weights(p):
    wb = p['Wbeta']
    wba = jnp.tile(wb[:H] + wb[2 * H:], (1, H))
    wbb = jnp.tile(wb[H:2 * H] - wb[2 * H:], (1, H))
    return wba, wbb


def kernel(x_token, x_phrase, params, ei_t2p, ei_p2t):
    pad0 = jnp.zeros((PE - E,), jnp.int32)
    padn = jnp.full((PE - E,), N, jnp.int32)

    def prep(ei):
        src = ei[0].astype(jnp.int32)
        dst = ei[1].astype(jnp.int32)
        return (jnp.concatenate([src, pad0]),
                jnp.concatenate([dst, pad0]),
                jnp.concatenate([dst, padn]))

    src_t2p, dstg_t2p, dsts_t2p = prep(ei_t2p)
    src_p2t, dstg_p2t, dsts_p2t = prep(ei_p2t)

    h_t = _embed(x_token, params['emb_W'], params['emb_b'].reshape(1, H))
    h_p = x_phrase

    pt = params['t2p']
    pp = params['p2t']
    wba_t, wbb_t = _gate_weights(pt)
    wba_p, wbb_p = _gate_weights(pp)

    a = _conv(h_t, h_p, src_t2p, dstg_t2p, dsts_t2p, pt)
    h_p2 = _epilogue_leaky(a, h_p, pt['Wskip'], pt['bskip'], wba_t, wbb_t)
    b = _conv(h_p, h_t, src_p2t, dstg_p2t, dsts_p2t, pp)
    h_t2 = _epilogue_leaky(b, h_t, pp['Wskip'], pp['bskip'], wba_p, wbb_p)
    c = _conv(h_p2, h_t2, src_p2t, dstg_p2t, dsts_p2t, pp)
    return _epilogue_head(c, h_t2, pp['Wskip'], pp['bskip'], wba_p, wbb_p,
                          params['head_W'], params['head_b'])
